# baseline (speedup 1.0000x reference)
import functools

import jax
import jax.numpy as jnp
from jax import lax
from jax.experimental import pallas as pl
from jax.experimental.pallas import tpu as pltpu
from jax.experimental.pallas import tpu_sc as plsc

_R, _N, _S = 4, 50000, 10
_B, _F, _H = 512, 256, 256
_SP = 16
_M = _B * _S
_RM = _R * _M
_NC, _NS = 2, 16
_NW = _NC * _NS
_PER_W = _RM // _NW
_WPR = _NW // _R
_BPW = _B // _WPR
_CH = 16
_NCH = _PER_W // _CH
_SPC = 2
_IPS = _CH * _S // _SPC
_IDXW = _PER_W * _S


def _sc_encode_gather(emb, neigh16, nodes):
    mesh = plsc.VectorSubcoreMesh(core_axis_name="c", subcore_axis_name="s")

    @functools.partial(
        pl.kernel,
        out_type=jax.ShapeDtypeStruct((_RM, _F), jnp.float32),
        mesh=mesh,
        scratch_types=[
            pltpu.VMEM((_BPW,), jnp.int32),
            pltpu.VMEM((_BPW, _SP), jnp.int32),
            pltpu.VMEM((_PER_W + 16,), jnp.int32),
            pltpu.VMEM((_PER_W, _SP), jnp.int32),
            pltpu.VMEM((_IDXW + 16,), jnp.int32),
            pltpu.VMEM((2, _CH * _S, _F), jnp.float32),
            pltpu.VMEM((2, _CH, _F), jnp.float32),
            pltpu.SemaphoreType.DMA,
            pltpu.SemaphoreType.DMA,
            pltpu.SemaphoreType.DMA,
            pltpu.SemaphoreType.DMA,
        ],
        compiler_params=pltpu.CompilerParams(use_tc_tiling_on_sc=False),
    )
    def k(emb_hbm, neigh_hbm, nodes_hbm, out_hbm,
          nodes_v, nb2_v, flat_v, h1_v, idx_all, rows_v, out_v,
          g0, g1, o0, o1):
        wid = lax.axis_index("s") * _NC + lax.axis_index("c")
        r = wid // _WPR
        rN = r * _N
        b0 = (wid % _WPR) * _BPW
        base = wid * _PER_W
        gsem = (g0, g1)
        osem = (o0, o1)

        pltpu.sync_copy(nodes_hbm.at[pl.ds(b0, _BPW)], nodes_v)
        for q in range(_BPW // 16):
            sl = pl.ds(q * 16, 16)
            nodes_v[sl] = nodes_v[sl] + rN
        pltpu.async_copy(neigh_hbm.at[nodes_v], nb2_v, g0).wait()

        def frow(i, carry):
            flat_v[pl.ds(i * _S, _SP)] = nb2_v[i, :] + rN
            return carry

        lax.fori_loop(0, _BPW, frow, 0)

        h1_cps = [
            pltpu.make_async_copy(
                neigh_hbm.at[flat_v.at[pl.ds(128 * j, 128)]],
                h1_v.at[pl.ds(128 * j, 128)],
                g0,
            )
            for j in range(_PER_W // 128)
        ]
        for cp in h1_cps:
            cp.start()
        for cp in h1_cps:
            cp.wait()

        def irow(i, carry):
            idx_all[pl.ds(i * _S, _SP)] = h1_v[i, :]
            return carry

        lax.fori_loop(0, _PER_W, irow, 0)

        def gather_cps(g, slot):
            return [
                pltpu.make_async_copy(
                    emb_hbm.at[idx_all.at[pl.ds(g * _CH * _S + j * _IPS, _IPS)]],
                    rows_v.at[slot, pl.ds(j * _IPS, _IPS)],
                    gsem[slot],
                )
                for j in range(_SPC)
            ]

        def out_cp(g, slot):
            return pltpu.make_async_copy(
                out_v.at[slot], out_hbm.at[pl.ds(base + g * _CH, _CH)], osem[slot]
            )

        def issue(g, slot):
            for cp in gather_cps(g, slot):
                cp.start()

        def compute(g, slot):
            @pl.when(g >= 2)
            def _drain():
                out_cp(g - 2, slot).wait()

            def row(i, c2):
                for f in range(_F // 16):
                    sl = pl.ds(f * 16, 16)
                    acc = rows_v[slot, i * _S, sl]
                    for s2 in range(1, _S):
                        acc = acc + rows_v[slot, i * _S + s2, sl]
                    out_v[slot, i, sl] = acc
                return c2

            lax.fori_loop(0, _CH, row, 0)
            out_cp(g, slot).start()

        issue(0, 0)

        def body2(h, carry):
            g = 2 * h
            for slot in range(2):
                gg = g + slot
                for cp in gather_cps(gg, slot):
                    cp.wait()

                @pl.when(gg + 1 < _NCH)
                def _next():
                    issue(gg + 1, 1 - slot)

                compute(gg, slot)
            return carry

        lax.fori_loop(0, _NCH // 2, body2, 0)
        out_cp(_NCH - 2, 0).wait()
        out_cp(_NCH - 1, 1).wait()

    return k(emb, neigh16, nodes)


def _tc_transform(agg, w1, w2):
    def body(a_ref, w1_ref, w2_ref, o_ref):
        r = pl.program_id(0)
        e = jnp.maximum(
            jnp.dot(a_ref[0], w1_ref[0], preferred_element_type=jnp.float32), 0.0
        )
        x = e.reshape(_B, _S, _H).sum(axis=1)
        h = jnp.maximum(
            jnp.dot(x, w2_ref[0], preferred_element_type=jnp.float32), 0.0
        )

        @pl.when(r == 0)
        def _init():
            o_ref[...] = h

        @pl.when(r != 0)
        def _acc():
            o_ref[...] += h

    return pl.pallas_call(
        body,
        grid=(_R,),
        in_specs=[
            pl.BlockSpec((1, _M, _F), lambda r: (r, 0, 0)),
            pl.BlockSpec((1, _F, _H), lambda r: (r, 0, 0)),
            pl.BlockSpec((1, _H, _H), lambda r: (r, 0, 0)),
        ],
        out_specs=pl.BlockSpec((_B, _H), lambda r: (0, 0)),
        out_shape=jax.ShapeDtypeStruct((_B, _H), jnp.float32),
    )(agg, w1, w2)


def kernel(emb, W1, W2, neigh, nodes):
    neigh16 = jnp.concatenate(
        [neigh.reshape(_R * _N, _S),
         jnp.zeros((_R * _N, _SP - _S), jnp.int32)],
        axis=1,
    )
    agg = _sc_encode_gather(emb, neigh16, nodes)
    inv_s = jnp.float32(1.0 / _S)
    return _tc_transform(agg.reshape(_R, _M, _F), W1 * inv_s, W2 * inv_s)


# speedup vs baseline: 1.1079x; 1.1079x over previous
import functools

import jax
import jax.numpy as jnp
from jax import lax
from jax.experimental import pallas as pl
from jax.experimental.pallas import tpu as pltpu
from jax.experimental.pallas import tpu_sc as plsc

_R, _N, _S = 4, 50000, 10
_B, _F, _H = 512, 256, 256
_SP = 16
_M = _B * _S
_RM = _R * _M
_NC, _NS = 2, 16
_NW = _NC * _NS
_PER_W = _RM // _NW
_WPR = _NW // _R
_BPW = _B // _WPR
_CH = 16
_NCH = _PER_W // _CH
_SPC = 2
_IPS = _CH * _S // _SPC
_IDXW = _PER_W * _S


def _wid():
    return lax.axis_index("s") * _NC + lax.axis_index("c")


def _sc_index_chain(neigh16, nodes):
    mesh = plsc.VectorSubcoreMesh(core_axis_name="c", subcore_axis_name="s")

    @functools.partial(
        pl.kernel,
        out_type=jax.ShapeDtypeStruct((_NW * _IDXW,), jnp.int32),
        mesh=mesh,
        scratch_types=[
            pltpu.VMEM((_BPW,), jnp.int32),
            pltpu.VMEM((_BPW, _SP), jnp.int32),
            pltpu.VMEM((_PER_W + 16,), jnp.int32),
            pltpu.VMEM((_PER_W, _SP), jnp.int32),
            pltpu.VMEM((_IDXW + 16,), jnp.int32),
            pltpu.SemaphoreType.DMA,
        ],
        compiler_params=pltpu.CompilerParams(use_tc_tiling_on_sc=False),
    )
    def k(neigh_hbm, nodes_hbm, idx_hbm,
          nodes_v, nb2_v, flat_v, h1_v, idx_all, sem):
        wid = _wid()
        rN = (wid // _WPR) * _N
        b0 = (wid % _WPR) * _BPW

        pltpu.sync_copy(nodes_hbm.at[pl.ds(b0, _BPW)], nodes_v)
        for q in range(_BPW // 16):
            sl = pl.ds(q * 16, 16)
            nodes_v[sl] = nodes_v[sl] + rN
        pltpu.async_copy(neigh_hbm.at[nodes_v], nb2_v, sem).wait()

        def frow(i, carry):
            flat_v[pl.ds(i * _S, _SP)] = nb2_v[i, :] + rN
            return carry

        lax.fori_loop(0, _BPW, frow, 0)

        h1_cps = [
            pltpu.make_async_copy(
                neigh_hbm.at[flat_v.at[pl.ds(128 * j, 128)]],
                h1_v.at[pl.ds(128 * j, 128)],
                sem,
            )
            for j in range(_PER_W // 128)
        ]
        for cp in h1_cps:
            cp.start()
        for cp in h1_cps:
            cp.wait()

        def irow(i, carry):
            idx_all[pl.ds(i * _S, _SP)] = h1_v[i, :]
            return carry

        lax.fori_loop(0, _PER_W, irow, 0)
        pltpu.sync_copy(idx_all.at[pl.ds(0, _IDXW)],
                        idx_hbm.at[pl.ds(wid * _IDXW, _IDXW)])

    return k(neigh16, nodes)


def _sc_gather_sum(emb, idx1):
    mesh = plsc.VectorSubcoreMesh(core_axis_name="c", subcore_axis_name="s")

    @functools.partial(
        pl.kernel,
        out_type=jax.ShapeDtypeStruct((_RM, _F), jnp.float32),
        mesh=mesh,
        scratch_types=[
            pltpu.VMEM((_IDXW,), jnp.int32),
            pltpu.VMEM((2, _CH * _S, _F), jnp.float32),
            pltpu.VMEM((2, _CH, _F), jnp.float32),
            pltpu.SemaphoreType.DMA,
            pltpu.SemaphoreType.DMA,
            pltpu.SemaphoreType.DMA,
            pltpu.SemaphoreType.DMA,
        ],
    )
    def k(emb_hbm, idx_hbm, out_hbm, idx_all, rows_v, out_v, g0, g1, o0, o1):
        wid = _wid()
        base = wid * _PER_W
        gsem = (g0, g1)
        osem = (o0, o1)

        pltpu.sync_copy(idx_hbm.at[pl.ds(wid * _IDXW, _IDXW)], idx_all)

        def gather_cps(g, slot):
            return [
                pltpu.make_async_copy(
                    emb_hbm.at[idx_all.at[pl.ds(g * _CH * _S + j * _IPS, _IPS)]],
                    rows_v.at[slot, pl.ds(j * _IPS, _IPS)],
                    gsem[slot],
                )
                for j in range(_SPC)
            ]

        def out_cp(g, slot):
            return pltpu.make_async_copy(
                out_v.at[slot], out_hbm.at[pl.ds(base + g * _CH, _CH)], osem[slot]
            )

        def issue(g, slot):
            for cp in gather_cps(g, slot):
                cp.start()

        def compute(g, slot):
            @pl.when(g >= 2)
            def _drain():
                out_cp(g - 2, slot).wait()

            def row(i, c2):
                for f in range(_F // 16):
                    sl = pl.ds(f * 16, 16)
                    acc = rows_v[slot, i * _S, sl]
                    for s2 in range(1, _S):
                        acc = acc + rows_v[slot, i * _S + s2, sl]
                    out_v[slot, i, sl] = acc
                return c2

            lax.fori_loop(0, _CH, row, 0)
            out_cp(g, slot).start()

        issue(0, 0)

        def body2(h, carry):
            g = 2 * h
            for slot in range(2):
                gg = g + slot
                for cp in gather_cps(gg, slot):
                    cp.wait()

                @pl.when(gg + 1 < _NCH)
                def _next():
                    issue(gg + 1, 1 - slot)

                compute(gg, slot)
            return carry

        lax.fori_loop(0, _NCH // 2, body2, 0)
        out_cp(_NCH - 2, 0).wait()
        out_cp(_NCH - 1, 1).wait()

    return k(emb, idx1)


def _tc_transform(agg, w1, w2):
    def body(a_ref, w1_ref, w2_ref, o_ref):
        r = pl.program_id(0)
        e = jnp.maximum(
            jnp.dot(a_ref[0], w1_ref[0], preferred_element_type=jnp.float32), 0.0
        )
        x = e.reshape(_B, _S, _H).sum(axis=1)
        h = jnp.maximum(
            jnp.dot(x, w2_ref[0], preferred_element_type=jnp.float32), 0.0
        )

        @pl.when(r == 0)
        def _init():
            o_ref[...] = h

        @pl.when(r != 0)
        def _acc():
            o_ref[...] += h

    return pl.pallas_call(
        body,
        grid=(_R,),
        in_specs=[
            pl.BlockSpec((1, _M, _F), lambda r: (r, 0, 0)),
            pl.BlockSpec((1, _F, _H), lambda r: (r, 0, 0)),
            pl.BlockSpec((1, _H, _H), lambda r: (r, 0, 0)),
        ],
        out_specs=pl.BlockSpec((_B, _H), lambda r: (0, 0)),
        out_shape=jax.ShapeDtypeStruct((_B, _H), jnp.float32),
    )(agg, w1, w2)


def kernel(emb, W1, W2, neigh, nodes):
    neigh16 = jnp.concatenate(
        [neigh.reshape(_R * _N, _S),
         jnp.zeros((_R * _N, _SP - _S), jnp.int32)],
        axis=1,
    )
    idx1 = _sc_index_chain(neigh16, nodes)
    agg = _sc_gather_sum(emb, idx1)
    inv_s = jnp.float32(1.0 / _S)
    return _tc_transform(agg.reshape(_R, _M, _F), W1 * inv_s, W2 * inv_s)
